# Initial kernel scaffold; baseline (speedup 1.0000x reference)
import jax
import jax.numpy as jnp
from jax.experimental import pallas as pl


def kernel(tokens, embedding):
    raise NotImplementedError("write your pallas kernel here")


# speedup vs baseline: 2.4148x; 2.4148x over previous
import functools
import jax
import jax.numpy as jnp
import numpy as np
from jax import lax
from jax.experimental import pallas as pl
from jax.experimental.pallas import tpu as pltpu
from jax.experimental.pallas import tpu_sc as plsc

VOCAB_D = 128
NC, NS, L = 2, 16, 16
NW = NC * NS
GROUP = 128
SCALE = np.float32(np.sqrt(np.float32(VOCAB_D)))


def _make_lookup(n_groups):
  mesh = plsc.VectorSubcoreMesh(
      core_axis_name="c", subcore_axis_name="s", num_cores=NC,
      num_subcores=NS)

  @functools.partial(
      pl.kernel,
      out_type=jax.ShapeDtypeStruct((NW, n_groups, GROUP, VOCAB_D),
                                    jnp.float32),
      mesh=mesh,
      scratch_types=[
          pltpu.VMEM((n_groups, GROUP), jnp.int32),
          pltpu.VMEM((GROUP, VOCAB_D), jnp.float32),
          pltpu.SemaphoreType.DMA,
      ],
  )
  def lookup(table_hbm, idx_hbm, out_hbm, idx_v, rows_v, sem):
    wid = lax.axis_index("s") * NC + lax.axis_index("c")
    pltpu.sync_copy(idx_hbm.at[wid], idx_v)

    def per_group(g, carry):
      pltpu.async_copy(table_hbm.at[idx_v.at[g]], rows_v, sem).wait()

      def per_row(r, c2):
        for c in range(VOCAB_D // L):
          rows_v[r, pl.ds(c * L, L)] = rows_v[r, pl.ds(c * L, L)] * SCALE
        return c2

      lax.fori_loop(0, GROUP, per_row, 0, unroll=2)
      pltpu.sync_copy(rows_v, out_hbm.at[wid, g])
      return carry

    lax.fori_loop(0, n_groups, per_group, 0)

  return lookup


def kernel(tokens, embedding):
  b, s = tokens.shape
  total = b * s
  assert total % (NW * GROUP) == 0
  n_groups = total // (NW * GROUP)
  idx = tokens.reshape(NW, n_groups, GROUP).astype(jnp.int32)
  out = _make_lookup(n_groups)(embedding, idx)
  return out.reshape(b, s, VOCAB_D)


# speedup vs baseline: 2.8781x; 1.1919x over previous
import functools
import jax
import jax.numpy as jnp
import numpy as np
from jax import lax
from jax.experimental import pallas as pl
from jax.experimental.pallas import tpu as pltpu
from jax.experimental.pallas import tpu_sc as plsc

VOCAB_D = 128
NC, NS, L = 2, 16, 16
NW = NC * NS
GROUP = 128
SCALE = np.float32(np.sqrt(np.float32(VOCAB_D)))


def _make_lookup(n_groups):
  assert n_groups % 2 == 0 and n_groups >= 4
  mesh = plsc.VectorSubcoreMesh(
      core_axis_name="c", subcore_axis_name="s", num_cores=NC,
      num_subcores=NS)

  @functools.partial(
      pl.kernel,
      out_type=jax.ShapeDtypeStruct((NW, n_groups, GROUP, VOCAB_D),
                                    jnp.float32),
      mesh=mesh,
      scratch_types=[
          pltpu.VMEM((n_groups, GROUP), jnp.int32),
          pltpu.VMEM((GROUP, VOCAB_D), jnp.float32),
          pltpu.VMEM((GROUP, VOCAB_D), jnp.float32),
          pltpu.SemaphoreType.DMA,
          pltpu.SemaphoreType.DMA,
      ],
  )
  def lookup(table_hbm, idx_hbm, out_hbm, idx_v, buf0, buf1, sem0, sem1):
    wid = lax.axis_index("s") * NC + lax.axis_index("c")
    pltpu.sync_copy(idx_hbm.at[wid], idx_v)
    bufs = (buf0, buf1)
    sems = (sem0, sem1)

    def gather_start(g, p):
      pltpu.async_copy(table_hbm.at[idx_v.at[g]], bufs[p], sems[p])

    def gather_wait(g, p):
      pltpu.make_async_copy(table_hbm.at[idx_v.at[g]], bufs[p],
                            sems[p]).wait()

    def scale_store(g, p):
      buf = bufs[p]

      def per_row(r, c2):
        for c in range(VOCAB_D // L):
          buf[r, pl.ds(c * L, L)] = buf[r, pl.ds(c * L, L)] * SCALE
        return c2

      lax.fori_loop(0, GROUP, per_row, 0, unroll=4)
      pltpu.sync_copy(buf, out_hbm.at[wid, g])

    gather_start(0, 0)

    def step(i, carry):
      g = i * 2
      gather_start(g + 1, 1)
      gather_wait(g, 0)
      scale_store(g, 0)
      gather_start(g + 2, 0)
      gather_wait(g + 1, 1)
      scale_store(g + 1, 1)
      return carry

    lax.fori_loop(0, n_groups // 2 - 1, step, 0)

    g = n_groups - 2
    gather_start(g + 1, 1)
    gather_wait(g, 0)
    scale_store(g, 0)
    gather_wait(g + 1, 1)
    scale_store(g + 1, 1)

  return lookup


def kernel(tokens, embedding):
  b, s = tokens.shape
  total = b * s
  assert total % (NW * GROUP) == 0
  n_groups = total // (NW * GROUP)
  idx = tokens.reshape(NW, n_groups, GROUP).astype(jnp.int32)
  out = _make_lookup(n_groups)(embedding, idx)
  return out.reshape(b, s, VOCAB_D)


# speedup vs baseline: 8.9912x; 3.1239x over previous
import functools
import jax
import jax.numpy as jnp
import numpy as np
from jax import lax
from jax.experimental import pallas as pl
from jax.experimental.pallas import tpu as pltpu
from jax.experimental.pallas import tpu_sc as plsc

VOCAB_D = 128
NC, NS, L = 2, 16, 16
NW = NC * NS
GROUP = 128
NBUF = 6
DEPTH = 3
SCALE = np.float32(np.sqrt(np.float32(VOCAB_D)))


def _make_lookup(n_groups):
  assert n_groups >= 2 * NBUF

  mesh = plsc.VectorSubcoreMesh(
      core_axis_name="c", subcore_axis_name="s", num_cores=NC,
      num_subcores=NS)

  @functools.partial(
      pl.kernel,
      out_type=jax.ShapeDtypeStruct((NW, n_groups, GROUP, VOCAB_D),
                                    jnp.float32),
      mesh=mesh,
      scratch_types=[
          pltpu.VMEM((n_groups, GROUP), jnp.int32),
          tuple(pltpu.VMEM((GROUP, VOCAB_D), jnp.float32)
                for _ in range(NBUF)),
          tuple(pltpu.SemaphoreType.DMA for _ in range(NBUF)),
          tuple(pltpu.SemaphoreType.DMA for _ in range(NBUF)),
      ],
  )
  def lookup(table_hbm, idx_hbm, out_hbm, idx_v, bufs, gsems, ssems):
    wid = lax.axis_index("s") * NC + lax.axis_index("c")
    pltpu.sync_copy(idx_hbm.at[wid], idx_v)

    def gather_start(g, p):
      pltpu.async_copy(table_hbm.at[idx_v.at[g]], bufs[p], gsems[p])

    def gather_wait(g, p):
      pltpu.make_async_copy(table_hbm.at[idx_v.at[g]], bufs[p],
                            gsems[p]).wait()

    def store_start(g, p):
      pltpu.async_copy(bufs[p], out_hbm.at[wid, g], ssems[p])

    def store_wait(g, p):
      pltpu.make_async_copy(bufs[p], out_hbm.at[wid, g], ssems[p]).wait()

    def scale(p):
      buf = bufs[p]

      def per_row(r, c2):
        for c in range(VOCAB_D // L):
          buf[r, pl.ds(c * L, L)] = buf[r, pl.ds(c * L, L)] * SCALE
        return c2

      lax.fori_loop(0, GROUP, per_row, 0, unroll=4)

    for k in range(DEPTH):
      gather_start(k, k)

    for k in range(NBUF):
      gather_wait(k, k)
      scale(k)
      store_start(k, k)
      if k < NBUF - DEPTH:
        gather_start(k + DEPTH, k + DEPTH)
      else:
        store_wait(k - (NBUF - DEPTH), (k + DEPTH) % NBUF)
        gather_start(k + DEPTH, (k + DEPTH) % NBUF)

    n_main = (n_groups - NBUF) // NBUF

    def step(i, carry):
      for k in range(NBUF):
        g = (i + 1) * NBUF + k
        gather_wait(g, k)
        scale(k)
        store_start(g, k)

        @pl.when(g + DEPTH < n_groups)
        def _():
          store_wait(g - (NBUF - DEPTH), (k + DEPTH) % NBUF)
          gather_start(g + DEPTH, (k + DEPTH) % NBUF)

      return carry

    lax.fori_loop(0, n_main, step, 0)

    for k in range(n_groups - NBUF - n_main * NBUF):
      g = NBUF + n_main * NBUF + k
      p = g % NBUF
      gather_wait(g, p)
      scale(p)
      store_start(g, p)

    for g in range(n_groups - NBUF, n_groups):
      store_wait(g, g % NBUF)

  return lookup


def kernel(tokens, embedding):
  b, s = tokens.shape
  total = b * s
  assert total % (NW * GROUP) == 0
  n_groups = total // (NW * GROUP)
  idx = tokens.T.reshape(NW, n_groups, GROUP).astype(jnp.int32)
  out = _make_lookup(n_groups)(embedding, idx)
  return out.reshape(s, b, VOCAB_D).transpose(1, 0, 2)


# speedup vs baseline: 8.9924x; 1.0001x over previous
import functools
import jax
import jax.numpy as jnp
import numpy as np
from jax import lax
from jax.experimental import pallas as pl
from jax.experimental.pallas import tpu as pltpu
from jax.experimental.pallas import tpu_sc as plsc

VOCAB_D = 128
NC, NS, L = 2, 16, 16
NW = NC * NS
GROUP = 128
NBUF = 7
DEPTH = 3
SCALE = np.float32(np.sqrt(np.float32(VOCAB_D)))


def _make_lookup(n_groups):
  assert n_groups >= 2 * NBUF

  mesh = plsc.VectorSubcoreMesh(
      core_axis_name="c", subcore_axis_name="s", num_cores=NC,
      num_subcores=NS)

  @functools.partial(
      pl.kernel,
      out_type=jax.ShapeDtypeStruct((NW, n_groups, GROUP, VOCAB_D),
                                    jnp.float32),
      mesh=mesh,
      scratch_types=[
          pltpu.VMEM((n_groups, GROUP), jnp.int32),
          tuple(pltpu.VMEM((GROUP, VOCAB_D), jnp.float32)
                for _ in range(NBUF)),
          tuple(pltpu.SemaphoreType.DMA for _ in range(NBUF)),
          tuple(pltpu.SemaphoreType.DMA for _ in range(NBUF)),
      ],
  )
  def lookup(table_hbm, idx_hbm, out_hbm, idx_v, bufs, gsems, ssems):
    wid = lax.axis_index("s") * NC + lax.axis_index("c")
    pltpu.sync_copy(idx_hbm.at[wid], idx_v)

    def gather_start(g, p):
      pltpu.async_copy(table_hbm.at[idx_v.at[g]], bufs[p], gsems[p])

    def gather_wait(g, p):
      pltpu.make_async_copy(table_hbm.at[idx_v.at[g]], bufs[p],
                            gsems[p]).wait()

    def store_start(g, p):
      pltpu.async_copy(bufs[p], out_hbm.at[wid, g], ssems[p])

    def store_wait(g, p):
      pltpu.make_async_copy(bufs[p], out_hbm.at[wid, g], ssems[p]).wait()

    def scale(p):
      buf = bufs[p]

      def per_row(r, c2):
        for c in range(VOCAB_D // L):
          buf[r, pl.ds(c * L, L)] = buf[r, pl.ds(c * L, L)] * SCALE
        return c2

      lax.fori_loop(0, GROUP, per_row, 0, unroll=4)

    for k in range(DEPTH):
      gather_start(k, k)

    for k in range(NBUF):
      gather_wait(k, k)
      scale(k)
      store_start(k, k)
      if k < NBUF - DEPTH:
        gather_start(k + DEPTH, k + DEPTH)
      else:
        store_wait(k - (NBUF - DEPTH), (k + DEPTH) % NBUF)
        gather_start(k + DEPTH, (k + DEPTH) % NBUF)

    n_main = (n_groups - NBUF) // NBUF

    def step(i, carry):
      for k in range(NBUF):
        g = (i + 1) * NBUF + k
        gather_wait(g, k)
        scale(k)
        store_start(g, k)

        @pl.when(g + DEPTH < n_groups)
        def _():
          store_wait(g - (NBUF - DEPTH), (k + DEPTH) % NBUF)
          gather_start(g + DEPTH, (k + DEPTH) % NBUF)

      return carry

    lax.fori_loop(0, n_main, step, 0)

    for k in range(n_groups - NBUF - n_main * NBUF):
      g = NBUF + n_main * NBUF + k
      p = g % NBUF
      gather_wait(g, p)
      scale(p)
      store_start(g, p)

    for g in range(n_groups - NBUF, n_groups):
      store_wait(g, g % NBUF)

  return lookup


def kernel(tokens, embedding):
  b, s = tokens.shape
  total = b * s
  assert total % (NW * GROUP) == 0
  n_groups = total // (NW * GROUP)
  idx = tokens.T.reshape(NW, n_groups, GROUP).astype(jnp.int32)
  out = _make_lookup(n_groups)(embedding, idx)
  return out.reshape(s, b, VOCAB_D).transpose(1, 0, 2)


# speedup vs baseline: 9.0535x; 1.0068x over previous
import functools
import jax
import jax.numpy as jnp
import numpy as np
from jax import lax
from jax.experimental import pallas as pl
from jax.experimental.pallas import tpu as pltpu
from jax.experimental.pallas import tpu_sc as plsc

VOCAB_D = 128
NC, NS, L = 2, 16, 16
NW = NC * NS
GROUP = 128
NBUF = 6
DEPTH = 3
SCALE = np.float32(np.sqrt(np.float32(VOCAB_D)))


def _make_lookup(n_groups):
  assert n_groups >= 2 * NBUF

  mesh = plsc.VectorSubcoreMesh(
      core_axis_name="c", subcore_axis_name="s", num_cores=NC,
      num_subcores=NS)

  @functools.partial(
      pl.kernel,
      out_type=jax.ShapeDtypeStruct((NW, n_groups, GROUP, VOCAB_D),
                                    jnp.float32),
      mesh=mesh,
      scratch_types=[
          pltpu.VMEM((n_groups, GROUP), jnp.int32),
          tuple(pltpu.VMEM((GROUP, VOCAB_D), jnp.float32)
                for _ in range(NBUF)),
          tuple(pltpu.SemaphoreType.DMA for _ in range(NBUF)),
          tuple(pltpu.SemaphoreType.DMA for _ in range(NBUF)),
      ],
  )
  def lookup(table_hbm, idx_hbm, out_hbm, idx_v, bufs, gsems, ssems):
    wid = lax.axis_index("s") * NC + lax.axis_index("c")
    pltpu.sync_copy(idx_hbm.at[wid], idx_v)

    def gather_start(g, p):
      pltpu.async_copy(table_hbm.at[idx_v.at[g]], bufs[p], gsems[p])

    def gather_wait(g, p):
      pltpu.make_async_copy(table_hbm.at[idx_v.at[g]], bufs[p],
                            gsems[p]).wait()

    def store_start(g, p):
      pltpu.async_copy(bufs[p], out_hbm.at[wid, g], ssems[p])

    def store_wait(g, p):
      pltpu.make_async_copy(bufs[p], out_hbm.at[wid, g], ssems[p]).wait()

    def scale(p):
      buf = bufs[p]

      def per_row(r, c2):
        for c in range(VOCAB_D // L):
          buf[r, pl.ds(c * L, L)] = buf[r, pl.ds(c * L, L)] * SCALE
        return c2

      lax.fori_loop(0, GROUP, per_row, 0, unroll=4)

    for k in range(DEPTH):
      gather_start(k, k)

    for k in range(NBUF):
      gather_wait(k, k)
      scale(k)
      store_start(k, k)
      if k < NBUF - DEPTH:
        gather_start(k + DEPTH, k + DEPTH)
      else:
        store_wait(k - (NBUF - DEPTH), (k + DEPTH) % NBUF)
        gather_start(k + DEPTH, (k + DEPTH) % NBUF)

    n_main = (n_groups - NBUF) // NBUF

    def step(i, carry):
      for k in range(NBUF):
        g = (i + 1) * NBUF + k
        gather_wait(g, k)
        scale(k)
        store_start(g, k)

        @pl.when(g + DEPTH < n_groups)
        def _():
          store_wait(g - (NBUF - DEPTH), (k + DEPTH) % NBUF)
          gather_start(g + DEPTH, (k + DEPTH) % NBUF)

      return carry

    lax.fori_loop(0, n_main, step, 0)

    for k in range(n_groups - NBUF - n_main * NBUF):
      g = NBUF + n_main * NBUF + k
      p = g % NBUF
      gather_wait(g, p)
      scale(p)
      store_start(g, p)

    for g in range(n_groups - NBUF, n_groups):
      store_wait(g, g % NBUF)

  return lookup


def kernel(tokens, embedding):
  b, s = tokens.shape
  total = b * s
  assert total % (NW * GROUP) == 0
  n_groups = total // (NW * GROUP)
  idx = tokens.T.reshape(NW, n_groups, GROUP).astype(jnp.int32)
  out = _make_lookup(n_groups)(embedding, idx)
  return out.reshape(s, b, VOCAB_D).transpose(1, 0, 2)
